# baseline (speedup 1.0000x reference)
import jax
import jax.numpy as jnp
from jax import lax
from jax.experimental import pallas as pl
from jax.experimental.pallas import tpu as pltpu
from jax.experimental.pallas import tpu_sc as plsc

B, L, C = 1024, 20, 128
NW = 32
BPW = B // NW
IPW = BPW * L
NV = (16, 64, 128)
TBASE = (0, NW * 16, NW * (16 + 64))
HPW = BPW // 2
RPP = HPW * L


def _sc_pool_body(bpw, idx_hbm, table, out_hbm, idx_v, rows0, rows1, out_v,
                  sem0, sem1):
    ipw = bpw * L
    rpp = ipw // 2
    nc = plsc.get_sparse_core_info().num_cores
    wid = lax.axis_index("s") * nc + lax.axis_index("c")

    pltpu.sync_copy(idx_hbm.at[pl.ds(wid * 3 * ipw, 3 * ipw)], idx_v)
    roffs = [(TBASE[f] + wid * NV[f]).astype(jnp.int32) for f in range(3)]
    for k in range(3 * ipw // 16):
        sl16 = pl.ds(k * 16, 16)
        idx_v[sl16] = idx_v[sl16] + roffs[k // (ipw // 16)]

    rows = (rows0, rows1)
    sems = (sem0, sem1)

    chunks = []
    o = 0
    while o < rpp:
        n = min(128, rpp - o)
        chunks.append((o, n))
        o += n

    def fire(p):
        f, h = divmod(p, 2)
        base = f * ipw + h * rpp
        buf, sem = rows[p % 2], sems[p % 2]
        return [
            pltpu.async_copy(table.at[idx_v.at[pl.ds(base + o, n)]],
                             buf.at[pl.ds(o, n)], sem)
            for o, n in chunks
        ]

    pending = fire(0)
    for p in range(6):
        nxt = fire(p + 1) if p < 5 else []
        for cp in pending:
            cp.wait()
        pending = nxt

        f, h = divmod(p, 2)
        buf = rows[p % 2]
        off = f * C

        def body(e, carry, buf=buf, off=off, h=h):
            base = e * L
            for c in range(C // 16):
                sl = pl.ds(16 * c, 16)
                acc = buf[base, sl]
                for l in range(1, L):
                    acc = acc + buf[base + l, sl]
                out_v[h * (bpw // 2) + e, pl.ds(off + 16 * c, 16)] = \
                    acc * (1.0 / L)
            return carry

        lax.fori_loop(0, bpw // 2, body, 0)

    pltpu.sync_copy(out_v, out_hbm.at[pl.ds(wid * bpw, bpw)])


def _sc_pool(idx_all, table_all, nb):
    import functools
    bpw = B // nb // NW
    ipw = bpw * L
    mesh = plsc.VectorSubcoreMesh(core_axis_name="c", subcore_axis_name="s")
    return pl.kernel(
        functools.partial(_sc_pool_body, bpw),
        out_type=jax.ShapeDtypeStruct((B // nb, 3 * C), jnp.float32),
        mesh=mesh,
        scratch_types=[
            pltpu.VMEM((3 * ipw,), jnp.int32),
            pltpu.VMEM((ipw // 2, C), jnp.float32),
            pltpu.VMEM((ipw // 2, C), jnp.float32),
            pltpu.VMEM((bpw, 3 * C), jnp.float32),
            pltpu.SemaphoreType.DMA,
            pltpu.SemaphoreType.DMA,
        ],
    )(idx_all, table_all)


def _silu(x):
    return x * jax.nn.sigmoid(x)


def _mm(x, w):
    return jnp.dot(x, w, preferred_element_type=jnp.float32,
                   precision=lax.Precision.HIGHEST)


def _tc_dense_body(emb_ref, eps_ref, cW1, cb1, cW2, cb2, gW1, gb1, gW2, gb2,
                   sW1, sb1, sW2, sb2, muW1, mub1, muW2, mub2, vW1, vb1,
                   vW2, vb2, out_ref):
    emb = emb_ref[...]
    ec = _silu(_mm(_silu(_mm(emb[:, 0 * C:1 * C], cW1[...]) + cb1[...]),
                   cW2[...]) + cb2[...])
    eg = _silu(_mm(_silu(_mm(emb[:, 1 * C:2 * C], gW1[...]) + gb1[...]),
                   gW2[...]) + gb2[...])
    es = _silu(_mm(_silu(_mm(emb[:, 2 * C:3 * C], sW1[...]) + sb1[...]),
                   sW2[...]) + sb2[...])
    cat = jnp.concatenate([ec, eg, es], axis=1)
    mu = _mm(jax.nn.relu(_mm(cat, muW1[...]) + mub1[...]), muW2[...]) \
        + mub2[...]
    var = _mm(jax.nn.relu(_mm(cat, vW1[...]) + vb1[...]), vW2[...]) \
        + vb2[...]
    out_ref[...] = mu + jnp.exp(0.5 * var) * eps_ref[...]


def _tc_dense(emb, eps, weights):
    blk = 256
    grid = emb.shape[0] // blk
    row_spec = lambda w: pl.BlockSpec((blk, w), lambda i: (i, 0))
    full = lambda a: pl.BlockSpec(a.shape, lambda i: (0,) * a.ndim)
    return pl.pallas_call(
        _tc_dense_body,
        grid=(grid,),
        in_specs=[row_spec(3 * C), row_spec(C)] + [full(w) for w in weights],
        out_specs=row_spec(C),
        out_shape=jax.ShapeDtypeStruct((emb.shape[0], C), jnp.float32),
    )(emb, eps, *weights)


def kernel(category, genre, style, cat_table, gen_table, sty_table,
           cW1, cb1, cW2, cb2, gW1, gb1, gW2, gb2, sW1, sb1, sW2, sb2,
           muW1, mub1, muW2, mub2, vW1, vb1, vW2, vb2, eps):
    nb = 2
    bpw = B // nb // NW
    ipw = bpw * L
    idx_all = jnp.stack([category.reshape(nb, NW, ipw),
                         genre.reshape(nb, NW, ipw),
                         style.reshape(nb, NW, ipw)], axis=2).reshape(nb, -1)
    table_all = jnp.concatenate([jnp.tile(cat_table, (NW, 1)),
                                 jnp.tile(gen_table, (NW, 1)),
                                 jnp.tile(sty_table, (NW, 1))], axis=0)
    weights = [cW1, cb1.reshape(1, -1), cW2, cb2.reshape(1, -1),
               gW1, gb1.reshape(1, -1), gW2, gb2.reshape(1, -1),
               sW1, sb1.reshape(1, -1), sW2, sb2.reshape(1, -1),
               muW1, mub1.reshape(1, -1), muW2, mub2.reshape(1, -1),
               vW1, vb1.reshape(1, -1), vW2, vb2.reshape(1, -1)]
    zs = []
    csz = B // nb
    for h in range(nb):
        emb_h = _sc_pool(idx_all[h], table_all, nb)
        zs.append(_tc_dense(emb_h, eps[h * csz:(h + 1) * csz], weights))
    return jnp.concatenate(zs, axis=0)


# speedup vs baseline: 1.1784x; 1.1784x over previous
import jax
import jax.numpy as jnp
from jax import lax
from jax.experimental import pallas as pl
from jax.experimental.pallas import tpu as pltpu
from jax.experimental.pallas import tpu_sc as plsc

B, L, C = 1024, 20, 128
NW = 32
BPW = B // NW
IPW = BPW * L
NV = (16, 64, 128)
TBASE = (0, NW * 16, NW * (16 + 64))
HPW = BPW // 2
RPP = HPW * L


def _sc_pool_body(bpw, idx_hbm, table, out_hbm, idx_v, rows0, rows1, out_v,
                  sem0, sem1):
    ipw = bpw * L
    rpp = ipw // 2
    nc = plsc.get_sparse_core_info().num_cores
    wid = lax.axis_index("s") * nc + lax.axis_index("c")

    pltpu.sync_copy(idx_hbm.at[pl.ds(wid * 3 * ipw, 3 * ipw)], idx_v)

    rows = (rows0, rows1)
    sems = (sem0, sem1)

    chunks = []
    o = 0
    while o < rpp:
        n = min(128, rpp - o)
        chunks.append((o, n))
        o += n

    def fire(p):
        f, h = divmod(p, 2)
        base = f * ipw + h * rpp
        buf, sem = rows[p % 2], sems[p % 2]
        return [
            pltpu.async_copy(table.at[idx_v.at[pl.ds(base + o, n)]],
                             buf.at[pl.ds(o, n)], sem)
            for o, n in chunks
        ]

    pending = fire(0)
    for p in range(6):
        nxt = fire(p + 1) if p < 5 else []
        for cp in pending:
            cp.wait()
        pending = nxt

        f, h = divmod(p, 2)
        buf = rows[p % 2]
        off = f * C

        def body(e, carry, buf=buf, off=off, h=h):
            base = e * L
            for c in range(C // 16):
                sl = pl.ds(16 * c, 16)
                acc = buf[base, sl]
                for l in range(1, L):
                    acc = acc + buf[base + l, sl]
                out_v[h * (bpw // 2) + e, pl.ds(off + 16 * c, 16)] = \
                    acc * (1.0 / L)
            return carry

        lax.fori_loop(0, bpw // 2, body, 0)

    pltpu.sync_copy(out_v, out_hbm.at[pl.ds(wid * bpw, bpw)])


def _sc_pool(idx_all, table_all, nb):
    import functools
    bpw = B // nb // NW
    ipw = bpw * L
    mesh = plsc.VectorSubcoreMesh(core_axis_name="c", subcore_axis_name="s")
    return pl.kernel(
        functools.partial(_sc_pool_body, bpw),
        out_type=jax.ShapeDtypeStruct((B // nb, 3 * C), jnp.float32),
        mesh=mesh,
        scratch_types=[
            pltpu.VMEM((3 * ipw,), jnp.int32),
            pltpu.VMEM((ipw // 2, C), jnp.float32),
            pltpu.VMEM((ipw // 2, C), jnp.float32),
            pltpu.VMEM((bpw, 3 * C), jnp.float32),
            pltpu.SemaphoreType.DMA,
            pltpu.SemaphoreType.DMA,
        ],
    )(idx_all, table_all)


def _silu(x):
    return x * jax.nn.sigmoid(x)


def _mm(x, w):
    return jnp.dot(x, w, preferred_element_type=jnp.float32,
                   precision=lax.Precision.HIGHEST)


def _tc_dense_body(emb_ref, eps_ref, cW1, cb1, cW2, cb2, gW1, gb1, gW2, gb2,
                   sW1, sb1, sW2, sb2, muW1, mub1, muW2, mub2, vW1, vb1,
                   vW2, vb2, out_ref):
    emb = emb_ref[...]
    ec = _silu(_mm(_silu(_mm(emb[:, 0 * C:1 * C], cW1[...]) + cb1[...]),
                   cW2[...]) + cb2[...])
    eg = _silu(_mm(_silu(_mm(emb[:, 1 * C:2 * C], gW1[...]) + gb1[...]),
                   gW2[...]) + gb2[...])
    es = _silu(_mm(_silu(_mm(emb[:, 2 * C:3 * C], sW1[...]) + sb1[...]),
                   sW2[...]) + sb2[...])
    cat = jnp.concatenate([ec, eg, es], axis=1)
    mu = _mm(jax.nn.relu(_mm(cat, muW1[...]) + mub1[...]), muW2[...]) \
        + mub2[...]
    var = _mm(jax.nn.relu(_mm(cat, vW1[...]) + vb1[...]), vW2[...]) \
        + vb2[...]
    out_ref[...] = mu + jnp.exp(0.5 * var) * eps_ref[...]


def _tc_dense(emb, eps, weights):
    blk = 256
    grid = emb.shape[0] // blk
    row_spec = lambda w: pl.BlockSpec((blk, w), lambda i: (i, 0))
    full = lambda a: pl.BlockSpec(a.shape, lambda i: (0,) * a.ndim)
    return pl.pallas_call(
        _tc_dense_body,
        grid=(grid,),
        in_specs=[row_spec(3 * C), row_spec(C)] + [full(w) for w in weights],
        out_specs=row_spec(C),
        out_shape=jax.ShapeDtypeStruct((emb.shape[0], C), jnp.float32),
    )(emb, eps, *weights)


def kernel(category, genre, style, cat_table, gen_table, sty_table,
           cW1, cb1, cW2, cb2, gW1, gb1, gW2, gb2, sW1, sb1, sW2, sb2,
           muW1, mub1, muW2, mub2, vW1, vb1, vW2, vb2, eps):
    nb = 1
    bpw = B // nb // NW
    ipw = bpw * L
    woff = jnp.arange(NW, dtype=jnp.int32)[:, None]
    idx_all = jnp.stack(
        [category.reshape(nb, NW, ipw) + (TBASE[0] + woff * NV[0]),
         genre.reshape(nb, NW, ipw) + (TBASE[1] + woff * NV[1]),
         style.reshape(nb, NW, ipw) + (TBASE[2] + woff * NV[2])],
        axis=2).reshape(nb, -1)
    table_all = jnp.concatenate([jnp.tile(cat_table, (NW, 1)),
                                 jnp.tile(gen_table, (NW, 1)),
                                 jnp.tile(sty_table, (NW, 1))], axis=0)
    weights = [cW1, cb1.reshape(1, -1), cW2, cb2.reshape(1, -1),
               gW1, gb1.reshape(1, -1), gW2, gb2.reshape(1, -1),
               sW1, sb1.reshape(1, -1), sW2, sb2.reshape(1, -1),
               muW1, mub1.reshape(1, -1), muW2, mub2.reshape(1, -1),
               vW1, vb1.reshape(1, -1), vW2, vb2.reshape(1, -1)]
    zs = []
    csz = B // nb
    for h in range(nb):
        emb_h = _sc_pool(idx_all[h], table_all, nb)
        zs.append(_tc_dense(emb_h, eps[h * csz:(h + 1) * csz], weights))
    return jnp.concatenate(zs, axis=0)


# speedup vs baseline: 1.1854x; 1.0059x over previous
import jax
import jax.numpy as jnp
from jax import lax
from jax.experimental import pallas as pl
from jax.experimental.pallas import tpu as pltpu
from jax.experimental.pallas import tpu_sc as plsc

B, L, C = 1024, 20, 128
NW = 32
BPW = B // NW
IPW = BPW * L
NV = (16, 64, 128)
TBASE = (0, NW * 16, NW * (16 + 64))
HPW = BPW // 2
RPP = HPW * L


def _sc_pool_body(bpw, idx_hbm, table, out_hbm, idx_v, rows0, rows1, out_v,
                  sem0, sem1):
    ipw = bpw * L
    rpp = ipw // 2
    nc = plsc.get_sparse_core_info().num_cores
    wid = lax.axis_index("s") * nc + lax.axis_index("c")

    pltpu.sync_copy(idx_hbm.at[pl.ds(wid * 3 * ipw, 3 * ipw)], idx_v)

    rows = (rows0, rows1)
    sems = (sem0, sem1)

    chunks = []
    o = 0
    while o < rpp:
        n = min(64, rpp - o)
        chunks.append((o, n))
        o += n

    def fire(p):
        f, h = divmod(p, 2)
        base = f * ipw + h * rpp
        buf, sem = rows[p % 2], sems[p % 2]
        return [
            pltpu.async_copy(table.at[idx_v.at[pl.ds(base + o, n)]],
                             buf.at[pl.ds(o, n)], sem)
            for o, n in chunks
        ]

    pending = fire(0)
    for p in range(6):
        nxt = fire(p + 1) if p < 5 else []
        for cp in pending:
            cp.wait()
        pending = nxt

        f, h = divmod(p, 2)
        buf = rows[p % 2]
        off = f * C

        def body(e, carry, buf=buf, off=off, h=h):
            base = e * L
            for c in range(C // 16):
                sl = pl.ds(16 * c, 16)
                acc = buf[base, sl]
                for l in range(1, L):
                    acc = acc + buf[base + l, sl]
                out_v[h * (bpw // 2) + e, pl.ds(off + 16 * c, 16)] = \
                    acc * (1.0 / L)
            return carry

        lax.fori_loop(0, bpw // 2, body, 0)

    pltpu.sync_copy(out_v, out_hbm.at[pl.ds(wid * bpw, bpw)])


def _sc_pool(idx_all, table_all, nb):
    import functools
    bpw = B // nb // NW
    ipw = bpw * L
    mesh = plsc.VectorSubcoreMesh(core_axis_name="c", subcore_axis_name="s")
    return pl.kernel(
        functools.partial(_sc_pool_body, bpw),
        out_type=jax.ShapeDtypeStruct((B // nb, 3 * C), jnp.float32),
        mesh=mesh,
        scratch_types=[
            pltpu.VMEM((3 * ipw,), jnp.int32),
            pltpu.VMEM((ipw // 2, C), jnp.float32),
            pltpu.VMEM((ipw // 2, C), jnp.float32),
            pltpu.VMEM((bpw, 3 * C), jnp.float32),
            pltpu.SemaphoreType.DMA,
            pltpu.SemaphoreType.DMA,
        ],
    )(idx_all, table_all)


def _silu(x):
    return x * jax.nn.sigmoid(x)


def _mm(x, w):
    return jnp.dot(x, w, preferred_element_type=jnp.float32,
                   precision=lax.Precision.HIGHEST)


def _tc_dense_body(emb_ref, eps_ref, cW1, cb1, cW2, cb2, gW1, gb1, gW2, gb2,
                   sW1, sb1, sW2, sb2, muW1, mub1, muW2, mub2, vW1, vb1,
                   vW2, vb2, out_ref):
    emb = emb_ref[...]
    ec = _silu(_mm(_silu(_mm(emb[:, 0 * C:1 * C], cW1[...]) + cb1[...]),
                   cW2[...]) + cb2[...])
    eg = _silu(_mm(_silu(_mm(emb[:, 1 * C:2 * C], gW1[...]) + gb1[...]),
                   gW2[...]) + gb2[...])
    es = _silu(_mm(_silu(_mm(emb[:, 2 * C:3 * C], sW1[...]) + sb1[...]),
                   sW2[...]) + sb2[...])
    cat = jnp.concatenate([ec, eg, es], axis=1)
    mu = _mm(jax.nn.relu(_mm(cat, muW1[...]) + mub1[...]), muW2[...]) \
        + mub2[...]
    var = _mm(jax.nn.relu(_mm(cat, vW1[...]) + vb1[...]), vW2[...]) \
        + vb2[...]
    out_ref[...] = mu + jnp.exp(0.5 * var) * eps_ref[...]


def _tc_dense(emb, eps, weights):
    blk = 256
    grid = emb.shape[0] // blk
    row_spec = lambda w: pl.BlockSpec((blk, w), lambda i: (i, 0))
    full = lambda a: pl.BlockSpec(a.shape, lambda i: (0,) * a.ndim)
    return pl.pallas_call(
        _tc_dense_body,
        grid=(grid,),
        in_specs=[row_spec(3 * C), row_spec(C)] + [full(w) for w in weights],
        out_specs=row_spec(C),
        out_shape=jax.ShapeDtypeStruct((emb.shape[0], C), jnp.float32),
    )(emb, eps, *weights)


def kernel(category, genre, style, cat_table, gen_table, sty_table,
           cW1, cb1, cW2, cb2, gW1, gb1, gW2, gb2, sW1, sb1, sW2, sb2,
           muW1, mub1, muW2, mub2, vW1, vb1, vW2, vb2, eps):
    nb = 1
    bpw = B // nb // NW
    ipw = bpw * L
    woff = jnp.arange(NW, dtype=jnp.int32)[:, None]
    idx_all = jnp.stack(
        [category.reshape(nb, NW, ipw) + (TBASE[0] + woff * NV[0]),
         genre.reshape(nb, NW, ipw) + (TBASE[1] + woff * NV[1]),
         style.reshape(nb, NW, ipw) + (TBASE[2] + woff * NV[2])],
        axis=2).reshape(nb, -1)
    table_all = jnp.concatenate([jnp.tile(cat_table, (NW, 1)),
                                 jnp.tile(gen_table, (NW, 1)),
                                 jnp.tile(sty_table, (NW, 1))], axis=0)
    weights = [cW1, cb1.reshape(1, -1), cW2, cb2.reshape(1, -1),
               gW1, gb1.reshape(1, -1), gW2, gb2.reshape(1, -1),
               sW1, sb1.reshape(1, -1), sW2, sb2.reshape(1, -1),
               muW1, mub1.reshape(1, -1), muW2, mub2.reshape(1, -1),
               vW1, vb1.reshape(1, -1), vW2, vb2.reshape(1, -1)]
    zs = []
    csz = B // nb
    for h in range(nb):
        emb_h = _sc_pool(idx_all[h], table_all, nb)
        zs.append(_tc_dense(emb_h, eps[h * csz:(h + 1) * csz], weights))
    return jnp.concatenate(zs, axis=0)


# speedup vs baseline: 1.3124x; 1.1072x over previous
import jax
import jax.numpy as jnp
from jax import lax
from jax.experimental import pallas as pl
from jax.experimental.pallas import tpu as pltpu
from jax.experimental.pallas import tpu_sc as plsc

B, L, C = 1024, 20, 128
NW = 32
BPW = B // NW
IPW = BPW * L
NV = (64, 128)
TBASE = (0, NW * 64)
HPW = BPW // 2
RPP = HPW * L


def _sc_pool_body(bpw, idx_hbm, table, out_hbm, idx_v, rows0, rows1, out_v,
                  sem0, sem1):
    ipw = bpw * L
    rpp = ipw // 2
    nc = plsc.get_sparse_core_info().num_cores
    wid = lax.axis_index("s") * nc + lax.axis_index("c")

    pltpu.sync_copy(idx_hbm.at[pl.ds(wid * 2 * ipw, 2 * ipw)], idx_v)

    rows = (rows0, rows1)
    sems = (sem0, sem1)

    chunks = []
    o = 0
    while o < rpp:
        n = min(64, rpp - o)
        chunks.append((o, n))
        o += n

    def fire(p):
        f, h = divmod(p, 2)
        base = f * ipw + h * rpp
        buf, sem = rows[p % 2], sems[p % 2]
        return [
            pltpu.async_copy(table.at[idx_v.at[pl.ds(base + o, n)]],
                             buf.at[pl.ds(o, n)], sem)
            for o, n in chunks
        ]

    pending = fire(0)
    for p in range(4):
        nxt = fire(p + 1) if p < 3 else []
        for cp in pending:
            cp.wait()
        pending = nxt

        f, h = divmod(p, 2)
        buf = rows[p % 2]
        off = f * C

        def body(e, carry, buf=buf, off=off, h=h):
            base = e * L
            for c in range(C // 16):
                sl = pl.ds(16 * c, 16)
                acc = buf[base, sl]
                for l in range(1, L):
                    acc = acc + buf[base + l, sl]
                out_v[h * (bpw // 2) + e, pl.ds(off + 16 * c, 16)] = \
                    acc * (1.0 / L)
            return carry

        lax.fori_loop(0, bpw // 2, body, 0)

    pltpu.sync_copy(out_v, out_hbm.at[pl.ds(wid * bpw, bpw)])


def _sc_pool(idx_all, table_all, nb):
    import functools
    bpw = B // nb // NW
    ipw = bpw * L
    mesh = plsc.VectorSubcoreMesh(core_axis_name="c", subcore_axis_name="s")
    return pl.kernel(
        functools.partial(_sc_pool_body, bpw),
        out_type=jax.ShapeDtypeStruct((B // nb, 2 * C), jnp.float32),
        mesh=mesh,
        scratch_types=[
            pltpu.VMEM((2 * ipw,), jnp.int32),
            pltpu.VMEM((ipw // 2, C), jnp.float32),
            pltpu.VMEM((ipw // 2, C), jnp.float32),
            pltpu.VMEM((bpw, 2 * C), jnp.float32),
            pltpu.SemaphoreType.DMA,
            pltpu.SemaphoreType.DMA,
        ],
    )(idx_all, table_all)


def _silu(x):
    return x * jax.nn.sigmoid(x)


def _mm(x, w):
    return jnp.dot(x, w, preferred_element_type=jnp.float32,
                   precision=lax.Precision.HIGHEST)


def _tc_dense_body(emb_ref, eps_ref, cat_ref, cat_t, cW1, cb1, cW2, cb2,
                   gW1, gb1, gW2, gb2, sW1, sb1, sW2, sb2, muW1, mub1,
                   muW2, mub2, vW1, vb1, vW2, vb2, out_ref):
    emb = emb_ref[...]
    cat_blk = cat_ref[...]
    iota = lax.broadcasted_iota(jnp.int32, (1, 16), 1)
    cnt = jnp.zeros((cat_blk.shape[0], 16), jnp.float32)
    for l in range(L):
        cnt = cnt + (cat_blk[:, l:l + 1] == iota).astype(jnp.float32)
    ecp = _mm(cnt, cat_t[...]) * (1.0 / L)
    ec = _silu(_mm(_silu(_mm(ecp, cW1[...]) + cb1[...]),
                   cW2[...]) + cb2[...])
    eg = _silu(_mm(_silu(_mm(emb[:, 0 * C:1 * C], gW1[...]) + gb1[...]),
                   gW2[...]) + gb2[...])
    es = _silu(_mm(_silu(_mm(emb[:, 1 * C:2 * C], sW1[...]) + sb1[...]),
                   sW2[...]) + sb2[...])
    cat = jnp.concatenate([ec, eg, es], axis=1)
    mu = _mm(jax.nn.relu(_mm(cat, muW1[...]) + mub1[...]), muW2[...]) \
        + mub2[...]
    var = _mm(jax.nn.relu(_mm(cat, vW1[...]) + vb1[...]), vW2[...]) \
        + vb2[...]
    out_ref[...] = mu + jnp.exp(0.5 * var) * eps_ref[...]


def _tc_dense(emb, eps, category, cat_table, weights):
    blk = 256
    grid = emb.shape[0] // blk
    row_spec = lambda w: pl.BlockSpec((blk, w), lambda i: (i, 0))
    full = lambda a: pl.BlockSpec(a.shape, lambda i: (0,) * a.ndim)
    return pl.pallas_call(
        _tc_dense_body,
        grid=(grid,),
        in_specs=[row_spec(2 * C), row_spec(C), row_spec(L),
                  full(cat_table)] + [full(w) for w in weights],
        out_specs=row_spec(C),
        out_shape=jax.ShapeDtypeStruct((emb.shape[0], C), jnp.float32),
    )(emb, eps, category, cat_table, *weights)


def kernel(category, genre, style, cat_table, gen_table, sty_table,
           cW1, cb1, cW2, cb2, gW1, gb1, gW2, gb2, sW1, sb1, sW2, sb2,
           muW1, mub1, muW2, mub2, vW1, vb1, vW2, vb2, eps):
    nb = 1
    bpw = B // nb // NW
    ipw = bpw * L
    woff = jnp.arange(NW, dtype=jnp.int32)[:, None]
    idx_all = jnp.stack(
        [genre.reshape(nb, NW, ipw) + (TBASE[0] + woff * NV[0]),
         style.reshape(nb, NW, ipw) + (TBASE[1] + woff * NV[1])],
        axis=2).reshape(nb, -1)
    table_all = jnp.concatenate([jnp.tile(gen_table, (NW, 1)),
                                 jnp.tile(sty_table, (NW, 1))], axis=0)
    weights = [cW1, cb1.reshape(1, -1), cW2, cb2.reshape(1, -1),
               gW1, gb1.reshape(1, -1), gW2, gb2.reshape(1, -1),
               sW1, sb1.reshape(1, -1), sW2, sb2.reshape(1, -1),
               muW1, mub1.reshape(1, -1), muW2, mub2.reshape(1, -1),
               vW1, vb1.reshape(1, -1), vW2, vb2.reshape(1, -1)]
    zs = []
    csz = B // nb
    for h in range(nb):
        emb_h = _sc_pool(idx_all[h], table_all, nb)
        zs.append(_tc_dense(emb_h, eps[h * csz:(h + 1) * csz],
                            category[h * csz:(h + 1) * csz], cat_table,
                            weights))
    return jnp.concatenate(zs, axis=0)


# speedup vs baseline: 1.5199x; 1.1581x over previous
import jax
import jax.numpy as jnp
from jax import lax
from jax.experimental import pallas as pl
from jax.experimental.pallas import tpu as pltpu
from jax.experimental.pallas import tpu_sc as plsc

B, L, C = 1024, 20, 128
NW = 32
BPW = B // NW
IPW = BPW * L
NV = (128,)
TBASE = (0,)
HPW = BPW // 2
RPP = HPW * L


def _sc_pool_body(bpw, idx_hbm, table, out_hbm, idx_v, rows0, rows1, out_v,
                  sem0, sem1):
    ipw = bpw * L
    rpp = ipw // 2
    nc = plsc.get_sparse_core_info().num_cores
    wid = lax.axis_index("s") * nc + lax.axis_index("c")

    pltpu.sync_copy(idx_hbm.at[pl.ds(wid * ipw, ipw)], idx_v)

    rows = (rows0, rows1)
    sems = (sem0, sem1)

    chunks = []
    o = 0
    while o < rpp:
        n = min(64, rpp - o)
        chunks.append((o, n))
        o += n

    def fire(p):
        f, h = divmod(p, 2)
        base = f * ipw + h * rpp
        buf, sem = rows[p % 2], sems[p % 2]
        return [
            pltpu.async_copy(table.at[idx_v.at[pl.ds(base + o, n)]],
                             buf.at[pl.ds(o, n)], sem)
            for o, n in chunks
        ]

    pending = fire(0)
    for p in range(2):
        nxt = fire(p + 1) if p < 1 else []
        for cp in pending:
            cp.wait()
        pending = nxt

        f, h = divmod(p, 2)
        buf = rows[p % 2]
        off = f * C

        def body(e, carry, buf=buf, off=off, h=h):
            base = e * L
            for c in range(C // 16):
                sl = pl.ds(16 * c, 16)
                acc = buf[base, sl]
                for l in range(1, L):
                    acc = acc + buf[base + l, sl]
                out_v[h * (bpw // 2) + e, pl.ds(off + 16 * c, 16)] = \
                    acc * (1.0 / L)
            return carry

        lax.fori_loop(0, bpw // 2, body, 0)

    pltpu.sync_copy(out_v, out_hbm.at[pl.ds(wid * bpw, bpw)])


def _sc_pool(idx_all, table_all, nb):
    import functools
    bpw = B // nb // NW
    ipw = bpw * L
    mesh = plsc.VectorSubcoreMesh(core_axis_name="c", subcore_axis_name="s")
    return pl.kernel(
        functools.partial(_sc_pool_body, bpw),
        out_type=jax.ShapeDtypeStruct((B // nb, C), jnp.float32),
        mesh=mesh,
        scratch_types=[
            pltpu.VMEM((ipw,), jnp.int32),
            pltpu.VMEM((ipw // 2, C), jnp.float32),
            pltpu.VMEM((ipw // 2, C), jnp.float32),
            pltpu.VMEM((bpw, C), jnp.float32),
            pltpu.SemaphoreType.DMA,
            pltpu.SemaphoreType.DMA,
        ],
    )(idx_all, table_all)


def _silu(x):
    return x * jax.nn.sigmoid(x)


def _mm(x, w):
    return jnp.dot(x, w, preferred_element_type=jnp.float32,
                   precision=lax.Precision.HIGHEST)


def _tc_dense_body(emb_ref, eps_ref, cat_ref, cat_t, gen_ref, gen_t,
                   cW1, cb1, cW2, cb2, gW1, gb1, gW2, gb2, sW1, sb1,
                   sW2, sb2, muW1, mub1, muW2, mub2, vW1, vb1, vW2, vb2,
                   out_ref):
    emb = emb_ref[...]

    def pooled(idx_ref, table_ref, nv):
        blk_idx = idx_ref[...]
        iota = lax.broadcasted_iota(jnp.int32, (1, nv), 1)
        cnt = jnp.zeros((blk_idx.shape[0], nv), jnp.float32)
        for l in range(L):
            cnt = cnt + (blk_idx[:, l:l + 1] == iota).astype(jnp.float32)
        return _mm(cnt, table_ref[...]) * (1.0 / L)

    ecp = pooled(cat_ref, cat_t, 16)
    egp = pooled(gen_ref, gen_t, 64)
    ec = _silu(_mm(_silu(_mm(ecp, cW1[...]) + cb1[...]),
                   cW2[...]) + cb2[...])
    eg = _silu(_mm(_silu(_mm(egp, gW1[...]) + gb1[...]),
                   gW2[...]) + gb2[...])
    es = _silu(_mm(_silu(_mm(emb[:, 0 * C:1 * C], sW1[...]) + sb1[...]),
                   sW2[...]) + sb2[...])
    cat = jnp.concatenate([ec, eg, es], axis=1)
    mu = _mm(jax.nn.relu(_mm(cat, muW1[...]) + mub1[...]), muW2[...]) \
        + mub2[...]
    var = _mm(jax.nn.relu(_mm(cat, vW1[...]) + vb1[...]), vW2[...]) \
        + vb2[...]
    out_ref[...] = mu + jnp.exp(0.5 * var) * eps_ref[...]


def _tc_dense(emb, eps, category, cat_table, genre, gen_table, weights):
    blk = 256
    grid = emb.shape[0] // blk
    row_spec = lambda w: pl.BlockSpec((blk, w), lambda i: (i, 0))
    full = lambda a: pl.BlockSpec(a.shape, lambda i: (0,) * a.ndim)
    return pl.pallas_call(
        _tc_dense_body,
        grid=(grid,),
        in_specs=[row_spec(C), row_spec(C), row_spec(L), full(cat_table),
                  row_spec(L), full(gen_table)] + [full(w) for w in weights],
        out_specs=row_spec(C),
        out_shape=jax.ShapeDtypeStruct((emb.shape[0], C), jnp.float32),
    )(emb, eps, category, cat_table, genre, gen_table, *weights)


def kernel(category, genre, style, cat_table, gen_table, sty_table,
           cW1, cb1, cW2, cb2, gW1, gb1, gW2, gb2, sW1, sb1, sW2, sb2,
           muW1, mub1, muW2, mub2, vW1, vb1, vW2, vb2, eps):
    nb = 1
    bpw = B // nb // NW
    ipw = bpw * L
    woff = jnp.arange(NW, dtype=jnp.int32)[:, None]
    idx_all = (style.reshape(nb, NW, ipw)
               + (TBASE[0] + woff * NV[0])).reshape(nb, -1)
    table_all = jnp.tile(sty_table, (NW, 1))
    weights = [cW1, cb1.reshape(1, -1), cW2, cb2.reshape(1, -1),
               gW1, gb1.reshape(1, -1), gW2, gb2.reshape(1, -1),
               sW1, sb1.reshape(1, -1), sW2, sb2.reshape(1, -1),
               muW1, mub1.reshape(1, -1), muW2, mub2.reshape(1, -1),
               vW1, vb1.reshape(1, -1), vW2, vb2.reshape(1, -1)]
    zs = []
    csz = B // nb
    for h in range(nb):
        emb_h = _sc_pool(idx_all[h], table_all, nb)
        zs.append(_tc_dense(emb_h, eps[h * csz:(h + 1) * csz],
                            category[h * csz:(h + 1) * csz], cat_table,
                            genre[h * csz:(h + 1) * csz], gen_table,
                            weights))
    return jnp.concatenate(zs, axis=0)


# speedup vs baseline: 1.5643x; 1.0292x over previous
import jax
import jax.numpy as jnp
from jax import lax
from jax.experimental import pallas as pl
from jax.experimental.pallas import tpu as pltpu
from jax.experimental.pallas import tpu_sc as plsc

B, L, C = 1024, 20, 128
NW = 32
BPW = B // NW
IPW = BPW * L
NV = (128,)
TBASE = (0,)
HPW = BPW // 2
RPP = HPW * L


def _sc_pool_body(bpw, idx_hbm, table, out_hbm, idx_v, rows0, rows1, out_v,
                  sem0, sem1):
    ipw = bpw * L
    rpp = ipw // 2
    nc = plsc.get_sparse_core_info().num_cores
    wid = lax.axis_index("s") * nc + lax.axis_index("c")

    pltpu.sync_copy(idx_hbm.at[pl.ds(wid * ipw, ipw)], idx_v)

    rows = (rows0, rows1)
    sems = (sem0, sem1)

    chunks = []
    o = 0
    while o < rpp:
        n = min(64, rpp - o)
        chunks.append((o, n))
        o += n

    def fire(p):
        f, h = divmod(p, 2)
        base = f * ipw + h * rpp
        buf, sem = rows[p % 2], sems[p % 2]
        return [
            pltpu.async_copy(table.at[idx_v.at[pl.ds(base + o, n)]],
                             buf.at[pl.ds(o, n)], sem)
            for o, n in chunks
        ]

    pending = fire(0)
    for p in range(2):
        nxt = fire(p + 1) if p < 1 else []
        for cp in pending:
            cp.wait()
        pending = nxt

        f, h = divmod(p, 2)
        buf = rows[p % 2]
        off = f * C

        def body(e, carry, buf=buf, off=off, h=h):
            base = e * L
            for c in range(C // 16):
                sl = pl.ds(16 * c, 16)
                acc = buf[base, sl]
                for l in range(1, L):
                    acc = acc + buf[base + l, sl]
                out_v[h * (bpw // 2) + e, pl.ds(off + 16 * c, 16)] = \
                    acc * (1.0 / L)
            return carry

        lax.fori_loop(0, bpw // 2, body, 0)

    pltpu.sync_copy(out_v, out_hbm.at[pl.ds(wid * bpw, bpw)])


def _sc_pool(idx_all, table_all, nb):
    import functools
    bpw = B // nb // NW
    ipw = bpw * L
    mesh = plsc.VectorSubcoreMesh(core_axis_name="c", subcore_axis_name="s")
    return pl.kernel(
        functools.partial(_sc_pool_body, bpw),
        out_type=jax.ShapeDtypeStruct((B // nb, C), jnp.float32),
        mesh=mesh,
        scratch_types=[
            pltpu.VMEM((ipw,), jnp.int32),
            pltpu.VMEM((ipw // 2, C), jnp.float32),
            pltpu.VMEM((ipw // 2, C), jnp.float32),
            pltpu.VMEM((bpw, C), jnp.float32),
            pltpu.SemaphoreType.DMA,
            pltpu.SemaphoreType.DMA,
        ],
    )(idx_all, table_all)


def _silu(x):
    return x * jax.nn.sigmoid(x)


def _mm(x, w):
    return jnp.dot(x, w, preferred_element_type=jnp.float32,
                   precision=lax.Precision.HIGHEST)


def _tc_dense_body(emb_ref, eps_ref, cat_ref, cat_t, gen_ref, gen_t,
                   cW1, cb1, cW2, cb2, gW1, gb1, gW2, gb2, sW1, sb1,
                   sW2, sb2, muW1, mub1, muW2, mub2, vW1, vb1, vW2, vb2,
                   out_ref):
    emb = emb_ref[...]

    def pooled(idx_ref, table_ref, nv):
        blk_idx = idx_ref[...]
        iota = lax.broadcasted_iota(jnp.int32, (1, nv), 1)
        cnt = jnp.zeros((blk_idx.shape[0], nv), jnp.float32)
        for l in range(L):
            cnt = cnt + (blk_idx[:, l:l + 1] == iota).astype(jnp.float32)
        return _mm(cnt, table_ref[...]) * (1.0 / L)

    ecp = pooled(cat_ref, cat_t, 16)
    egp = pooled(gen_ref, gen_t, 64)
    ec = _silu(_mm(_silu(_mm(ecp, cW1[...]) + cb1[...]),
                   cW2[...]) + cb2[...])
    eg = _silu(_mm(_silu(_mm(egp, gW1[...]) + gb1[...]),
                   gW2[...]) + gb2[...])
    es = _silu(_mm(_silu(_mm(emb[:, 0 * C:1 * C], sW1[...]) + sb1[...]),
                   sW2[...]) + sb2[...])
    cat = jnp.concatenate([ec, eg, es], axis=1)
    mu = _mm(jax.nn.relu(_mm(cat, muW1[...]) + mub1[...]), muW2[...]) \
        + mub2[...]
    var = _mm(jax.nn.relu(_mm(cat, vW1[...]) + vb1[...]), vW2[...]) \
        + vb2[...]
    out_ref[...] = mu + jnp.exp(0.5 * var) * eps_ref[...]


def _tc_dense(emb, eps, category, cat_table, genre, gen_table, weights):
    blk = 1024
    grid = emb.shape[0] // blk
    row_spec = lambda w: pl.BlockSpec((blk, w), lambda i: (i, 0))
    full = lambda a: pl.BlockSpec(a.shape, lambda i: (0,) * a.ndim)
    return pl.pallas_call(
        _tc_dense_body,
        grid=(grid,),
        in_specs=[row_spec(C), row_spec(C), row_spec(L), full(cat_table),
                  row_spec(L), full(gen_table)] + [full(w) for w in weights],
        out_specs=row_spec(C),
        out_shape=jax.ShapeDtypeStruct((emb.shape[0], C), jnp.float32),
    )(emb, eps, category, cat_table, genre, gen_table, *weights)


def kernel(category, genre, style, cat_table, gen_table, sty_table,
           cW1, cb1, cW2, cb2, gW1, gb1, gW2, gb2, sW1, sb1, sW2, sb2,
           muW1, mub1, muW2, mub2, vW1, vb1, vW2, vb2, eps):
    nb = 1
    bpw = B // nb // NW
    ipw = bpw * L
    woff = jnp.arange(NW, dtype=jnp.int32)[:, None]
    idx_all = (style.reshape(nb, NW, ipw)
               + (TBASE[0] + woff * NV[0])).reshape(nb, -1)
    table_all = jnp.tile(sty_table, (NW, 1))
    weights = [cW1, cb1.reshape(1, -1), cW2, cb2.reshape(1, -1),
               gW1, gb1.reshape(1, -1), gW2, gb2.reshape(1, -1),
               sW1, sb1.reshape(1, -1), sW2, sb2.reshape(1, -1),
               muW1, mub1.reshape(1, -1), muW2, mub2.reshape(1, -1),
               vW1, vb1.reshape(1, -1), vW2, vb2.reshape(1, -1)]
    zs = []
    csz = B // nb
    for h in range(nb):
        emb_h = _sc_pool(idx_all[h], table_all, nb)
        zs.append(_tc_dense(emb_h, eps[h * csz:(h + 1) * csz],
                            category[h * csz:(h + 1) * csz], cat_table,
                            genre[h * csz:(h + 1) * csz], gen_table,
                            weights))
    return jnp.concatenate(zs, axis=0)
